# baseline (speedup 1.0000x reference)
import functools

import jax
import jax.numpy as jnp
from jax import lax
from jax.experimental import pallas as pl
from jax.experimental.pallas import tpu as pltpu
from jax.experimental.pallas import tpu_sc as plsc

D = 1024
BM = 128

NC_SC, NS_SC = 2, 16
NW = NC_SC * NS_SC
CHUNK = 16
NBUF = 5

TB = 512
K_INT, K_TEST, K_TAG = 8, 1664, 1024


def _proj_body(a_ref, w_ref, o_ref):
    o_ref[...] = jnp.dot(a_ref[...].astype(jnp.bfloat16),
                         w_ref[...].astype(jnp.bfloat16),
                         preferred_element_type=jnp.float32)


def _project_table(table, w_comb, t):
    v = table.shape[0]
    return pl.pallas_call(
        _proj_body,
        grid=(pl.cdiv(v, BM),),
        in_specs=[pl.BlockSpec((BM, D), lambda i: (i, 0)),
                  pl.BlockSpec((D, D), lambda i: (t, 0))],
        out_specs=pl.BlockSpec((BM, D), lambda i: (i, 0)),
        out_shape=jax.ShapeDtypeStruct((v, D), jnp.float32),
    )(table, w_comb)


def _make_gather(rows):
    rpw = rows // NW
    nch = rpw // CHUNK
    mesh = plsc.VectorSubcoreMesh(core_axis_name="c", subcore_axis_name="s")
    buf_ty = pltpu.VMEM((CHUNK, D), jnp.float32)

    @functools.partial(
        pl.kernel,
        mesh=mesh,
        out_type=jax.ShapeDtypeStruct((rows, D), jnp.float32),
        scratch_types=[pltpu.VMEM((rpw,), jnp.int32)]
                      + [buf_ty] * NBUF
                      + [pltpu.SemaphoreType.DMA] * NBUF,
    )
    def gather(p_hbm, idx_hbm, out_hbm, idx_v, *bufs_sems):
        bufs, sems = bufs_sems[:NBUF], bufs_sems[NBUF:]
        wid = lax.axis_index("s") * NC_SC + lax.axis_index("c")
        base = wid * rpw
        pltpu.sync_copy(idx_hbm.at[pl.ds(base, rpw)], idx_v)

        def issue(ci, gbuf, sem):
            @pl.when(ci < nch)
            def _():
                pltpu.async_copy(
                    p_hbm.at[idx_v.at[pl.ds(ci * CHUNK, CHUNK)]], gbuf, sem)

        def wait(gbuf, sem):
            pltpu.make_async_copy(p_hbm.at[pl.ds(0, CHUNK)], gbuf,
                                  sem).wait()

        def writeback(ci, gbuf):
            pltpu.sync_copy(gbuf,
                            out_hbm.at[pl.ds(base + ci * CHUNK, CHUNK)])

        for c in range(NBUF):
            issue(c, bufs[c], sems[c])

        def ring(g, _):
            for b in range(NBUF):
                ci = NBUF * g + b
                wait(bufs[b], sems[b])
                writeback(ci, bufs[b])
                issue(ci + NBUF, bufs[b], sems[b])
            return 0

        lax.fori_loop(0, nch // NBUF, ring, 0)

    return gather


def _onehot_body(ii_ref, it_ref, ig_ref, p0_ref, p2_ref, p3_ref, o_ref):
    def onehot(iref, k):
        idx = iref[0, 0, :]
        cols = lax.broadcasted_iota(jnp.int32, (TB, k), 1)
        return (cols == idx[:, None]).astype(jnp.bfloat16)

    acc = jnp.dot(onehot(ii_ref, K_INT), p0_ref[...],
                  preferred_element_type=jnp.float32)
    acc += jnp.dot(onehot(it_ref, K_TEST), p2_ref[...],
                   preferred_element_type=jnp.float32)
    acc += jnp.dot(onehot(ig_ref, K_TAG), p3_ref[...],
                   preferred_element_type=jnp.float32)
    o_ref[...] = acc


def _onehot_partial(ii, it, ig, p0, p2, p3, rows):
    nb = rows // TB
    iblk = pl.BlockSpec((1, 1, TB), lambda i: (i, 0, 0))
    return pl.pallas_call(
        _onehot_body,
        grid=(nb,),
        in_specs=[iblk, iblk, iblk,
                  pl.BlockSpec((K_INT, D), lambda i: (0, 0)),
                  pl.BlockSpec((K_TEST, D), lambda i: (0, 0)),
                  pl.BlockSpec((K_TAG, D), lambda i: (0, 0))],
        out_specs=pl.BlockSpec((TB, D), lambda i: (i, 0)),
        out_shape=jax.ShapeDtypeStruct((rows, D), jnp.float32),
    )(ii.reshape(nb, 1, TB), it.reshape(nb, 1, TB), ig.reshape(nb, 1, TB),
      p0, p2, p3)


def _make_ln_body(rb, seq):
    def _ln_body(q_ref, p_ref, b_ref, g_ref, bb_ref, o_ref):
        x = q_ref[...] + p_ref[...] + b_ref[...]
        mu = jnp.mean(x, axis=1, keepdims=True)
        xc = x - mu
        var = jnp.mean(xc * xc, axis=1, keepdims=True)
        y = xc * lax.rsqrt(var + 1e-6) * g_ref[...] + bb_ref[...]
        for j in range(rb):
            o_ref[j] = y[j * seq:(j + 1) * seq, :]
    return _ln_body


def _layernorm(qrows, partial, b, g, bb, bsz, seq):
    rb = 16
    bl = rb * seq
    vec = pl.BlockSpec((1, D), lambda i: (0, 0))
    blk = pl.BlockSpec((bl, D), lambda i: (i, 0))
    return pl.pallas_call(
        _make_ln_body(rb, seq),
        grid=(bsz // rb,),
        in_specs=[blk, blk, vec, vec, vec],
        out_specs=pl.BlockSpec((rb, seq, D), lambda i: (i, 0, 0)),
        out_shape=jax.ShapeDtypeStruct((bsz, seq, D), jnp.float32),
    )(qrows, partial, b.reshape(1, D), g.reshape(1, D), bb.reshape(1, D))


def kernel(testId, assessmentItemID, KnowledgeTag, answerCode, mask,
           interaction, emb_interaction, emb_test, emb_question, emb_tag,
           W_comb, b_comb, ln_g, ln_b):
    bsz, seq = interaction.shape
    rows = bsz * seq

    p_int = _project_table(emb_interaction, W_comb, 0)
    p_test = _project_table(emb_test, W_comb, 1)
    p_q = _project_table(emb_question, W_comb, 2)
    p_tag = _project_table(emb_tag, W_comb, 3)

    q_idx = assessmentItemID.reshape(rows).astype(jnp.int32)
    qrows = _make_gather(rows)(p_q, q_idx)

    def padto(x, n):
        return jnp.pad(x, ((0, n - x.shape[0]), (0, 0)))

    partial = _onehot_partial(
        interaction.reshape(rows).astype(jnp.int32),
        testId.reshape(rows).astype(jnp.int32),
        KnowledgeTag.reshape(rows).astype(jnp.int32),
        padto(p_int.astype(jnp.bfloat16), K_INT),
        padto(p_test.astype(jnp.bfloat16), K_TEST),
        padto(p_tag.astype(jnp.bfloat16), K_TAG),
        rows)

    x = _layernorm(qrows, partial, b_comb, ln_g, ln_b, bsz, seq)
    return (x, bsz)


# speedup vs baseline: 1.0874x; 1.0874x over previous
import functools

import jax
import jax.numpy as jnp
from jax import lax
from jax.experimental import pallas as pl
from jax.experimental.pallas import tpu as pltpu
from jax.experimental.pallas import tpu_sc as plsc

D = 1024
BM = 128

NC_SC, NS_SC = 2, 16
NW = NC_SC * NS_SC
CHUNK = 16
NBUF = 5

TB = 1600
K_INT, K_TEST, K_TAG = 8, 1664, 1024


def _proj_body(a_ref, w_ref, o_ref):
    o_ref[...] = jnp.dot(a_ref[...].astype(jnp.bfloat16),
                         w_ref[...].astype(jnp.bfloat16),
                         preferred_element_type=jnp.float32)


def _project_table(table, w_comb, t):
    v = table.shape[0]
    return pl.pallas_call(
        _proj_body,
        grid=(pl.cdiv(v, BM),),
        in_specs=[pl.BlockSpec((BM, D), lambda i: (i, 0)),
                  pl.BlockSpec((D, D), lambda i: (t, 0))],
        out_specs=pl.BlockSpec((BM, D), lambda i: (i, 0)),
        out_shape=jax.ShapeDtypeStruct((v, D), jnp.float32),
    )(table, w_comb)


def _make_gather(rows):
    rpw = rows // NW
    nch = rpw // CHUNK
    mesh = plsc.VectorSubcoreMesh(core_axis_name="c", subcore_axis_name="s")
    buf_ty = pltpu.VMEM((CHUNK, D), jnp.float32)

    @functools.partial(
        pl.kernel,
        mesh=mesh,
        out_type=jax.ShapeDtypeStruct((rows, D), jnp.float32),
        scratch_types=[pltpu.VMEM((rpw,), jnp.int32)]
                      + [buf_ty] * NBUF
                      + [pltpu.SemaphoreType.DMA] * NBUF,
    )
    def gather(p_hbm, idx_hbm, out_hbm, idx_v, *bufs_sems):
        bufs, sems = bufs_sems[:NBUF], bufs_sems[NBUF:]
        wid = lax.axis_index("s") * NC_SC + lax.axis_index("c")
        base = wid * rpw
        pltpu.sync_copy(idx_hbm.at[pl.ds(base, rpw)], idx_v)

        def issue(ci, gbuf, sem):
            @pl.when(ci < nch)
            def _():
                pltpu.async_copy(
                    p_hbm.at[idx_v.at[pl.ds(ci * CHUNK, CHUNK)]], gbuf, sem)

        def wait(gbuf, sem):
            pltpu.make_async_copy(p_hbm.at[pl.ds(0, CHUNK)], gbuf,
                                  sem).wait()

        def writeback(ci, gbuf):
            pltpu.sync_copy(gbuf,
                            out_hbm.at[pl.ds(base + ci * CHUNK, CHUNK)])

        for c in range(NBUF):
            issue(c, bufs[c], sems[c])

        def ring(g, _):
            for b in range(NBUF):
                ci = NBUF * g + b
                wait(bufs[b], sems[b])
                writeback(ci, bufs[b])
                issue(ci + NBUF, bufs[b], sems[b])
            return 0

        lax.fori_loop(0, nch // NBUF, ring, 0)

    return gather


def _make_fused_body(rb, seq):
    def _fused_body(ii_ref, it_ref, ig_ref, q_ref, p0_ref, p2_ref, p3_ref,
                    b_ref, g_ref, bb_ref, o_ref):
        def onehot(iref, k):
            idx = iref[0, 0, :]
            cols = lax.broadcasted_iota(jnp.int32, (TB, k), 1)
            return (cols == idx[:, None]).astype(jnp.bfloat16)

        acc = jnp.dot(onehot(ii_ref, K_INT), p0_ref[...],
                      preferred_element_type=jnp.float32)
        acc += jnp.dot(onehot(it_ref, K_TEST), p2_ref[...],
                       preferred_element_type=jnp.float32)
        acc += jnp.dot(onehot(ig_ref, K_TAG), p3_ref[...],
                       preferred_element_type=jnp.float32)
        x = acc + q_ref[...] + b_ref[...]
        mu = jnp.mean(x, axis=1, keepdims=True)
        xc = x - mu
        var = jnp.mean(xc * xc, axis=1, keepdims=True)
        y = xc * lax.rsqrt(var + 1e-6) * g_ref[...] + bb_ref[...]
        for j in range(rb):
            o_ref[j] = y[j * seq:(j + 1) * seq, :]
    return _fused_body


def _fused_onehot_ln(ii, it, ig, qrows, p0, p2, p3, b, g, bb, bsz, seq):
    rows = bsz * seq
    nb = rows // TB
    rb = TB // seq
    iblk = pl.BlockSpec((1, 1, TB), lambda i: (i, 0, 0))
    vec = pl.BlockSpec((1, D), lambda i: (0, 0))
    return pl.pallas_call(
        _make_fused_body(rb, seq),
        grid=(nb,),
        in_specs=[iblk, iblk, iblk,
                  pl.BlockSpec((TB, D), lambda i: (i, 0)),
                  pl.BlockSpec((K_INT, D), lambda i: (0, 0)),
                  pl.BlockSpec((K_TEST, D), lambda i: (0, 0)),
                  pl.BlockSpec((K_TAG, D), lambda i: (0, 0)),
                  vec, vec, vec],
        out_specs=pl.BlockSpec((rb, seq, D), lambda i: (i, 0, 0)),
        out_shape=jax.ShapeDtypeStruct((bsz, seq, D), jnp.float32),
    )(ii.reshape(nb, 1, TB), it.reshape(nb, 1, TB), ig.reshape(nb, 1, TB),
      qrows, p0, p2, p3,
      b.reshape(1, D), g.reshape(1, D), bb.reshape(1, D))


def kernel(testId, assessmentItemID, KnowledgeTag, answerCode, mask,
           interaction, emb_interaction, emb_test, emb_question, emb_tag,
           W_comb, b_comb, ln_g, ln_b):
    bsz, seq = interaction.shape
    rows = bsz * seq

    p_int = _project_table(emb_interaction, W_comb, 0)
    p_test = _project_table(emb_test, W_comb, 1)
    p_q = _project_table(emb_question, W_comb, 2)
    p_tag = _project_table(emb_tag, W_comb, 3)

    q_idx = assessmentItemID.reshape(rows).astype(jnp.int32)
    qrows = _make_gather(rows)(p_q, q_idx)

    def padto(x, n):
        return jnp.pad(x, ((0, n - x.shape[0]), (0, 0)))

    x = _fused_onehot_ln(
        interaction.reshape(rows).astype(jnp.int32),
        testId.reshape(rows).astype(jnp.int32),
        KnowledgeTag.reshape(rows).astype(jnp.int32),
        qrows,
        padto(p_int.astype(jnp.bfloat16), K_INT),
        padto(p_test.astype(jnp.bfloat16), K_TEST),
        padto(p_tag.astype(jnp.bfloat16), K_TAG),
        b_comb, ln_g, ln_b, bsz, seq)
    return (x, bsz)
